# Initial kernel scaffold; baseline (speedup 1.0000x reference)
import jax
import jax.numpy as jnp
from jax.experimental import pallas as pl


def kernel(embedding, scope_enc, spec_enc, W_init, b_init, W_ih, W_hh, b_ih, b_hh, W_out, b_out, Wd, We, v_attn, tokens_in, tokens_out, pointer_ids, ptr_targets, pointer_mask):
    raise NotImplementedError("write your pallas kernel here")


# speedup vs baseline: 3.6703x; 3.6703x over previous
import jax
import jax.numpy as jnp
from jax.experimental import pallas as pl
from jax.experimental.pallas import tpu as pltpu

B, T, S, H, V, E = 32, 256, 32, 512, 128, 512
C = 2
BC = B // C
TB = 64
NT = T // TB


def _body(spec_ref, scope_ref, emb_ref, wx_ref, wo_ref, whh_ref, winit_ref,
          wout_ref, wd_ref, we_ref, bih_ref, bhh_ref, bout_ref, binit_ref,
          v_ref, tin_ref, tout_ref, pid_ref, ptg_ref, pmk_ref,
          out_ref,
          tok_tab, scope_proj, ok_scr, h_scr, xp_scr, o_scr, acc_scr):
    k = pl.program_id(1)

    @pl.when(k == 0)
    def _init():
        tok_tab[...] = jnp.dot(emb_ref[...], wx_ref[...],
                               preferred_element_type=jnp.float32)
        sc = scope_ref[...].reshape(S * BC, H)
        scope_proj[...] = jnp.dot(sc, wo_ref[...],
                                  preferred_element_type=jnp.float32)
        ok_scr[...] = jnp.dot(sc, we_ref[...],
                              preferred_element_type=jnp.float32
                              ).reshape(S, BC, H)
        h_scr[...] = (jnp.dot(spec_ref[0], winit_ref[...],
                              preferred_element_type=jnp.float32)
                      + binit_ref[...])
        acc_scr[...] = jnp.zeros_like(acc_scr)

    tin = tin_ref[...].reshape(TB, BC)
    pid = pid_ref[...].reshape(TB, BC)
    pmk = pmk_ref[...].reshape(TB, BC)

    oh_tok = jnp.where(
        tin[..., None] == jax.lax.broadcasted_iota(jnp.int32, (TB, BC, V), 2),
        1.0, 0.0)
    xp = jnp.dot(oh_tok.reshape(TB * BC, V), tok_tab[...],
                 preferred_element_type=jnp.float32)
    bidx = jax.lax.broadcasted_iota(jnp.int32, (TB, BC), 1)
    col = jnp.where(pmk != 0, pid * BC + bidx, -1)
    oh_obj = jnp.where(
        col[..., None] == jax.lax.broadcasted_iota(jnp.int32,
                                                   (TB, BC, S * BC), 2),
        1.0, 0.0)
    xp = xp + jnp.dot(oh_obj.reshape(TB * BC, S * BC), scope_proj[...],
                      preferred_element_type=jnp.float32)
    xp_scr[...] = (xp + bih_ref[...]).reshape(TB, BC, 3 * H)

    def step(t, carry):
        xpt = xp_scr[t]
        h = h_scr[...]
        gh = jnp.dot(h, whh_ref[...],
                     preferred_element_type=jnp.float32) + bhh_ref[...]
        r = jax.nn.sigmoid(xpt[:, :H] + gh[:, :H])
        z = jax.nn.sigmoid(xpt[:, H:2 * H] + gh[:, H:2 * H])
        n = jnp.tanh(xpt[:, 2 * H:] + r * gh[:, 2 * H:])
        h_new = (1.0 - z) * n + z * h
        h_scr[...] = h_new
        o_scr[t] = h_new
        return carry

    jax.lax.fori_loop(0, TB, step, 0)

    o2 = o_scr[...].reshape(TB * BC, H)

    logits = jnp.dot(o2, wout_ref[...],
                     preferred_element_type=jnp.float32) + bout_ref[...]
    m = jnp.max(logits, axis=-1, keepdims=True)
    lse = jnp.log(jnp.sum(jnp.exp(logits - m), axis=-1, keepdims=True)) + m
    tout = tout_ref[...].reshape(TB, BC)
    oh_out = jnp.where(
        tout[..., None] == jax.lax.broadcasted_iota(jnp.int32, (TB, BC, V), 2),
        1.0, 0.0)
    sel = jnp.sum(oh_out.reshape(TB * BC, V) * logits, axis=-1, keepdims=True)
    tok_ll = (sel - lse).reshape(TB, BC, 1)

    hd = jnp.dot(o2, wd_ref[...],
                 preferred_element_type=jnp.float32).reshape(TB, BC, H)
    v = v_ref[...][None]
    parts = []
    for s in range(S):
        ts = jnp.tanh(hd + ok_scr[s][None])
        parts.append(jnp.sum(ts * v, axis=-1, keepdims=True))
    att = jnp.concatenate(parts, axis=-1)
    ms = jnp.max(att, axis=-1, keepdims=True)
    lse_s = jnp.log(jnp.sum(jnp.exp(att - ms), axis=-1, keepdims=True)) + ms
    ptg = jnp.where(pmk != 0, ptg_ref[...].reshape(TB, BC), -1)
    oh_tg = jnp.where(
        ptg[..., None] == jax.lax.broadcasted_iota(jnp.int32, (TB, BC, S), 2),
        1.0, 0.0)
    sel_s = jnp.sum(oh_tg * att, axis=-1, keepdims=True)
    mask_f = jnp.sum(oh_tg, axis=-1, keepdims=True)
    ptr_ll = sel_s - lse_s * mask_f

    acc_scr[...] = acc_scr[...] + jnp.sum(tok_ll + ptr_ll, axis=0)

    @pl.when(k == NT - 1)
    def _fin():
        out_ref[...] = acc_scr[...].reshape(1, BC, 128)


def kernel(embedding, scope_enc, spec_enc, W_init, b_init, W_ih, W_hh, b_ih,
           b_hh, W_out, b_out, Wd, We, v_attn, tokens_in, tokens_out,
           pointer_ids, ptr_targets, pointer_mask):
    f32 = jnp.float32
    wx = W_ih[:, :H].T
    wo = W_ih[:, H:].T
    whh = W_hh.T
    winit = W_init.T
    wout = W_out.T
    wd = Wd.T
    we = We.T
    spec2 = spec_enc.reshape(C, BC, E)
    scope_t = scope_enc.reshape(C, BC, S, H).transpose(0, 2, 1, 3)

    def ints(a):
        return (a.astype(jnp.int32).reshape(C, BC, NT, TB)
                .transpose(0, 2, 3, 1))

    b2 = lambda x: x.reshape(1, -1).astype(f32)
    full = lambda shp: pl.BlockSpec(shp, lambda c, k: tuple(0 for _ in shp))
    intspec = pl.BlockSpec((1, 1, TB, BC), lambda c, k: (c, k, 0, 0))

    out = pl.pallas_call(
        _body,
        grid=(C, NT),
        in_specs=[
            pl.BlockSpec((1, BC, E), lambda c, k: (c, 0, 0)),
            pl.BlockSpec((1, S, BC, H), lambda c, k: (c, 0, 0, 0)),
            full((V, H)),
            full((H, 3 * H)),
            full((H, 3 * H)),
            full((H, 3 * H)),
            full((E, H)),
            full((H, V)),
            full((H, H)),
            full((H, H)),
            full((1, 3 * H)),
            full((1, 3 * H)),
            full((1, V)),
            full((1, H)),
            full((1, H)),
            intspec, intspec, intspec, intspec, intspec,
        ],
        out_specs=pl.BlockSpec((1, BC, 128), lambda c, k: (c, 0, 0)),
        out_shape=jax.ShapeDtypeStruct((C, BC, 128), f32),
        scratch_shapes=[
            pltpu.VMEM((V, 3 * H), f32),
            pltpu.VMEM((S * BC, 3 * H), f32),
            pltpu.VMEM((S, BC, H), f32),
            pltpu.VMEM((BC, H), f32),
            pltpu.VMEM((TB, BC, 3 * H), f32),
            pltpu.VMEM((TB, BC, H), f32),
            pltpu.VMEM((BC, 128), f32),
        ],
        compiler_params=pltpu.CompilerParams(
            dimension_semantics=("parallel", "arbitrary"),
            vmem_limit_bytes=56 * 1024 * 1024,
        ),
        name="no_execution_fused",
    )(spec2, scope_t, embedding, wx, wo, whh, winit, wout, wd, we,
      b2(b_ih), b2(b_hh), b2(b_out), b2(b_init), b2(v_attn),
      ints(tokens_in), ints(tokens_out), ints(pointer_ids),
      ints(ptr_targets), ints(pointer_mask))
    return out.reshape(B, 128)[:, 0]


# speedup vs baseline: 3.8904x; 1.0600x over previous
import jax
import jax.numpy as jnp
from jax.experimental import pallas as pl
from jax.experimental.pallas import tpu as pltpu

B, T, S, H, V, E = 32, 256, 32, 512, 128, 512
C = 2
BC = B // C
TB = 64
NT = T // TB


def _body(spec_ref, scope_ref, emb_ref, wx_ref, wo_ref, whh_ref, winit_ref,
          wout_ref, wd_ref, we_ref, bih_ref, bhh_ref, bout_ref, binit_ref,
          v_ref, tin_ref, tout_ref, pid_ref, ptg_ref, pmk_ref,
          out_ref,
          tok_tab, scope_proj, ok_scr, h_scr, xp_scr, o_scr, acc_scr):
    k = pl.program_id(1)

    @pl.when(k == 0)
    def _init():
        tok_tab[...] = jnp.dot(emb_ref[...], wx_ref[...],
                               preferred_element_type=jnp.float32)
        sc = scope_ref[...].reshape(S * BC, H)
        scope_proj[...] = jnp.dot(sc, wo_ref[...],
                                  preferred_element_type=jnp.float32)
        ok_scr[...] = jnp.dot(sc, we_ref[...],
                              preferred_element_type=jnp.float32
                              ).reshape(S, BC, H)
        h_scr[...] = (jnp.dot(spec_ref[0], winit_ref[...],
                              preferred_element_type=jnp.float32)
                      + binit_ref[...])
        acc_scr[...] = jnp.zeros_like(acc_scr)

    tin = tin_ref[...].reshape(TB, BC)
    pid = pid_ref[...].reshape(TB, BC)
    pmk = pmk_ref[...].reshape(TB, BC)

    oh_tok = jnp.where(
        tin[..., None] == jax.lax.broadcasted_iota(jnp.int32, (TB, BC, V), 2),
        1.0, 0.0)
    xp = jnp.dot(oh_tok.reshape(TB * BC, V), tok_tab[...],
                 preferred_element_type=jnp.float32)
    bidx = jax.lax.broadcasted_iota(jnp.int32, (TB, BC), 1)
    col = jnp.where(pmk != 0, pid * BC + bidx, -1)
    oh_obj = jnp.where(
        col[..., None] == jax.lax.broadcasted_iota(jnp.int32,
                                                   (TB, BC, S * BC), 2),
        1.0, 0.0)
    xp = xp + jnp.dot(oh_obj.reshape(TB * BC, S * BC), scope_proj[...],
                      preferred_element_type=jnp.float32)
    xp_scr[...] = (xp + bih_ref[...]).reshape(TB, BC, 3 * H)

    def step(t, carry):
        xpt = xp_scr[t]
        h = h_scr[...]
        gh = jnp.dot(h.astype(jnp.bfloat16), whh_ref[...],
                     preferred_element_type=jnp.float32) + bhh_ref[...]
        r = jax.nn.sigmoid(xpt[:, :H] + gh[:, :H])
        z = jax.nn.sigmoid(xpt[:, H:2 * H] + gh[:, H:2 * H])
        n = jnp.tanh(xpt[:, 2 * H:] + r * gh[:, 2 * H:])
        h_new = (1.0 - z) * n + z * h
        h_scr[...] = h_new
        o_scr[t] = h_new
        return carry

    jax.lax.fori_loop(0, TB, step, 0, unroll=4)

    o2 = o_scr[...].reshape(TB * BC, H)

    logits = jnp.dot(o2, wout_ref[...],
                     preferred_element_type=jnp.float32) + bout_ref[...]
    m = jnp.max(logits, axis=-1, keepdims=True)
    lse = jnp.log(jnp.sum(jnp.exp(logits - m), axis=-1, keepdims=True)) + m
    tout = tout_ref[...].reshape(TB, BC)
    oh_out = jnp.where(
        tout[..., None] == jax.lax.broadcasted_iota(jnp.int32, (TB, BC, V), 2),
        1.0, 0.0)
    sel = jnp.sum(oh_out.reshape(TB * BC, V) * logits, axis=-1, keepdims=True)
    tok_ll = (sel - lse).reshape(TB, BC, 1)

    hd = jnp.dot(o2, wd_ref[...],
                 preferred_element_type=jnp.float32).reshape(TB, BC, H)
    v = v_ref[...][None]
    parts = []
    for s in range(S):
        ts = jnp.tanh(hd + ok_scr[s][None])
        parts.append(jnp.sum(ts * v, axis=-1, keepdims=True))
    att = jnp.concatenate(parts, axis=-1)
    ms = jnp.max(att, axis=-1, keepdims=True)
    lse_s = jnp.log(jnp.sum(jnp.exp(att - ms), axis=-1, keepdims=True)) + ms
    ptg = jnp.where(pmk != 0, ptg_ref[...].reshape(TB, BC), -1)
    oh_tg = jnp.where(
        ptg[..., None] == jax.lax.broadcasted_iota(jnp.int32, (TB, BC, S), 2),
        1.0, 0.0)
    sel_s = jnp.sum(oh_tg * att, axis=-1, keepdims=True)
    mask_f = jnp.sum(oh_tg, axis=-1, keepdims=True)
    ptr_ll = sel_s - lse_s * mask_f

    acc_scr[...] = acc_scr[...] + jnp.sum(tok_ll + ptr_ll, axis=0)

    @pl.when(k == NT - 1)
    def _fin():
        out_ref[...] = acc_scr[...].reshape(1, BC, 128)


def kernel(embedding, scope_enc, spec_enc, W_init, b_init, W_ih, W_hh, b_ih,
           b_hh, W_out, b_out, Wd, We, v_attn, tokens_in, tokens_out,
           pointer_ids, ptr_targets, pointer_mask):
    f32 = jnp.float32
    wx = W_ih[:, :H].T
    wo = W_ih[:, H:].T
    whh = W_hh.T.astype(jnp.bfloat16)
    winit = W_init.T
    wout = W_out.T
    wd = Wd.T
    we = We.T
    spec2 = spec_enc.reshape(C, BC, E)
    scope_t = scope_enc.reshape(C, BC, S, H).transpose(0, 2, 1, 3)

    def ints(a):
        return (a.astype(jnp.int32).reshape(C, BC, NT, TB)
                .transpose(0, 2, 3, 1))

    b2 = lambda x: x.reshape(1, -1).astype(f32)
    full = lambda shp: pl.BlockSpec(shp, lambda c, k: tuple(0 for _ in shp))
    intspec = pl.BlockSpec((1, 1, TB, BC), lambda c, k: (c, k, 0, 0))

    out = pl.pallas_call(
        _body,
        grid=(C, NT),
        in_specs=[
            pl.BlockSpec((1, BC, E), lambda c, k: (c, 0, 0)),
            pl.BlockSpec((1, S, BC, H), lambda c, k: (c, 0, 0, 0)),
            full((V, H)),
            full((H, 3 * H)),
            full((H, 3 * H)),
            full((H, 3 * H)),
            full((E, H)),
            full((H, V)),
            full((H, H)),
            full((H, H)),
            full((1, 3 * H)),
            full((1, 3 * H)),
            full((1, V)),
            full((1, H)),
            full((1, H)),
            intspec, intspec, intspec, intspec, intspec,
        ],
        out_specs=pl.BlockSpec((1, BC, 128), lambda c, k: (c, 0, 0)),
        out_shape=jax.ShapeDtypeStruct((C, BC, 128), f32),
        scratch_shapes=[
            pltpu.VMEM((V, 3 * H), f32),
            pltpu.VMEM((S * BC, 3 * H), f32),
            pltpu.VMEM((S, BC, H), f32),
            pltpu.VMEM((BC, H), f32),
            pltpu.VMEM((TB, BC, 3 * H), f32),
            pltpu.VMEM((TB, BC, H), f32),
            pltpu.VMEM((BC, 128), f32),
        ],
        compiler_params=pltpu.CompilerParams(
            dimension_semantics=("parallel", "arbitrary"),
            vmem_limit_bytes=56 * 1024 * 1024,
        ),
        name="no_execution_fused",
    )(spec2, scope_t, embedding, wx, wo, whh, winit, wout, wd, we,
      b2(b_ih), b2(b_hh), b2(b_out), b2(b_init), b2(v_attn),
      ints(tokens_in), ints(tokens_out), ints(pointer_ids),
      ints(ptr_targets), ints(pointer_mask))
    return out.reshape(B, 128)[:, 0]


# speedup vs baseline: 5.0945x; 1.3095x over previous
import jax
import jax.numpy as jnp
from jax.experimental import pallas as pl
from jax.experimental.pallas import tpu as pltpu

B, T, S, H, V, E = 32, 256, 32, 512, 128, 512
TB = 32
NT = T // TB
HB = 16


def _body(spec_ref, sc0_ref, sc1_ref, emb_ref, wx_ref, wo_ref, whh_ref,
          winit_ref, wout_ref, wd_ref, we_ref, bih_ref, bhh_ref, bout_ref,
          binit_ref, v_ref, tin_ref, tout_ref, pid_ref, ptg_ref, pmk_ref,
          out_ref,
          tok_tab, sp0, sp1, ok_scr, h_scr, xp_scr, o_scr, acc_scr):
    k = pl.program_id(0)
    bf16 = jnp.bfloat16
    f32 = jnp.float32

    @pl.when(k == 0)
    def _init():
        tok_tab[...] = jnp.dot(emb_ref[...], wx_ref[...],
                               preferred_element_type=f32).astype(bf16)
        s0 = sc0_ref[...].reshape(S * HB, H)
        s1 = sc1_ref[...].reshape(S * HB, H)
        sp0[...] = jnp.dot(s0, wo_ref[...],
                           preferred_element_type=f32).astype(bf16)
        sp1[...] = jnp.dot(s1, wo_ref[...],
                           preferred_element_type=f32).astype(bf16)
        ok_scr[:, :HB, :] = jnp.dot(s0, we_ref[...],
                                    preferred_element_type=f32
                                    ).reshape(S, HB, H)
        ok_scr[:, HB:, :] = jnp.dot(s1, we_ref[...],
                                    preferred_element_type=f32
                                    ).reshape(S, HB, H)
        h_scr[...] = (jnp.dot(spec_ref[...], winit_ref[...],
                              preferred_element_type=f32) + binit_ref[...])
        acc_scr[...] = jnp.zeros_like(acc_scr)

    tin = tin_ref[...].reshape(TB, B)
    pid = pid_ref[...].reshape(TB, B)
    pmk = pmk_ref[...].reshape(TB, B)
    bidx = jax.lax.broadcasted_iota(jnp.int32, (TB, HB), 1)

    for i, sp_i in ((0, sp0), (1, sp1)):
        sl = slice(HB * i, HB * (i + 1))
        th = tin[:, sl]
        oh_tok = jnp.where(
            th[..., None] == jax.lax.broadcasted_iota(jnp.int32,
                                                      (TB, HB, V), 2),
            1.0, 0.0).astype(bf16)
        xp = jnp.dot(oh_tok.reshape(TB * HB, V), tok_tab[...],
                     preferred_element_type=f32)
        col = jnp.where(pmk[:, sl] != 0, pid[:, sl] * HB + bidx, -1)
        oh_obj = jnp.where(
            col[..., None] == jax.lax.broadcasted_iota(jnp.int32,
                                                       (TB, HB, S * HB), 2),
            1.0, 0.0).astype(bf16)
        xp = xp + jnp.dot(oh_obj.reshape(TB * HB, S * HB), sp_i[...],
                          preferred_element_type=f32)
        xp_scr[:, sl, :] = (xp + bih_ref[...]).reshape(TB, HB, 3 * H)

    def step(t, carry):
        xpt = xp_scr[t]
        h = h_scr[...]
        gh = jnp.dot(h.astype(bf16), whh_ref[...],
                     preferred_element_type=f32) + bhh_ref[...]
        r = jax.nn.sigmoid(xpt[:, :H] + gh[:, :H])
        z = jax.nn.sigmoid(xpt[:, H:2 * H] + gh[:, H:2 * H])
        n = jnp.tanh(xpt[:, 2 * H:] + r * gh[:, 2 * H:])
        h_new = (1.0 - z) * n + z * h
        h_scr[...] = h_new
        o_scr[t] = h_new
        return carry

    jax.lax.fori_loop(0, TB, step, 0, unroll=4)

    o2 = o_scr[...].reshape(TB * B, H).astype(bf16)

    logits = jnp.dot(o2, wout_ref[...],
                     preferred_element_type=f32) + bout_ref[...]
    m = jnp.max(logits, axis=-1, keepdims=True)
    lse = jnp.log(jnp.sum(jnp.exp(logits - m), axis=-1, keepdims=True)) + m
    tout = tout_ref[...].reshape(TB, B)
    oh_out = jnp.where(
        tout[..., None] == jax.lax.broadcasted_iota(jnp.int32, (TB, B, V), 2),
        1.0, 0.0)
    sel = jnp.sum(oh_out.reshape(TB * B, V) * logits, axis=-1, keepdims=True)
    tok_ll = (sel - lse).reshape(TB, B, 1)

    hd = jnp.dot(o2, wd_ref[...],
                 preferred_element_type=f32).reshape(TB, B, H)
    v = v_ref[...][None]
    parts = []
    for s in range(S):
        ts = jnp.tanh(hd + ok_scr[s][None])
        parts.append(jnp.sum(ts * v, axis=-1, keepdims=True))
    att = jnp.concatenate(parts, axis=-1)
    ms = jnp.max(att, axis=-1, keepdims=True)
    lse_s = jnp.log(jnp.sum(jnp.exp(att - ms), axis=-1, keepdims=True)) + ms
    ptg = jnp.where(pmk != 0, ptg_ref[...].reshape(TB, B), -1)
    oh_tg = jnp.where(
        ptg[..., None] == jax.lax.broadcasted_iota(jnp.int32, (TB, B, S), 2),
        1.0, 0.0)
    sel_s = jnp.sum(oh_tg * att, axis=-1, keepdims=True)
    mask_f = jnp.sum(oh_tg, axis=-1, keepdims=True)
    ptr_ll = sel_s - lse_s * mask_f

    acc_scr[...] = acc_scr[...] + jnp.sum(tok_ll + ptr_ll, axis=0)

    @pl.when(k == NT - 1)
    def _fin():
        out_ref[...] = acc_scr[...]


def kernel(embedding, scope_enc, spec_enc, W_init, b_init, W_ih, W_hh, b_ih,
           b_hh, W_out, b_out, Wd, We, v_attn, tokens_in, tokens_out,
           pointer_ids, ptr_targets, pointer_mask):
    f32 = jnp.float32
    bf16 = jnp.bfloat16
    wx = W_ih[:, :H].T.astype(bf16)
    wo = W_ih[:, H:].T.astype(bf16)
    whh = W_hh.T.astype(bf16)
    winit = W_init.T.astype(bf16)
    wout = W_out.T.astype(bf16)
    wd = Wd.T.astype(bf16)
    we = We.T.astype(bf16)
    spec = spec_enc.astype(bf16)
    sc0 = scope_enc[:HB].transpose(1, 0, 2).astype(bf16)
    sc1 = scope_enc[HB:].transpose(1, 0, 2).astype(bf16)

    def ints(a):
        return a.astype(jnp.int32).reshape(B, NT, TB).transpose(1, 2, 0)

    b2 = lambda x: x.reshape(1, -1).astype(f32)
    full = lambda shp: pl.BlockSpec(shp, lambda k: tuple(0 for _ in shp))
    intspec = pl.BlockSpec((1, TB, B), lambda k: (k, 0, 0))

    out = pl.pallas_call(
        _body,
        grid=(NT,),
        in_specs=[
            full((B, E)),
            full((S, HB, H)),
            full((S, HB, H)),
            full((V, H)),
            full((H, 3 * H)),
            full((H, 3 * H)),
            full((H, 3 * H)),
            full((E, H)),
            full((H, V)),
            full((H, H)),
            full((H, H)),
            full((1, 3 * H)),
            full((1, 3 * H)),
            full((1, V)),
            full((1, H)),
            full((1, H)),
            intspec, intspec, intspec, intspec, intspec,
        ],
        out_specs=pl.BlockSpec((B, 128), lambda k: (0, 0)),
        out_shape=jax.ShapeDtypeStruct((B, 128), f32),
        scratch_shapes=[
            pltpu.VMEM((V, 3 * H), bf16),
            pltpu.VMEM((S * HB, 3 * H), bf16),
            pltpu.VMEM((S * HB, 3 * H), bf16),
            pltpu.VMEM((S, B, H), f32),
            pltpu.VMEM((B, H), f32),
            pltpu.VMEM((TB, B, 3 * H), f32),
            pltpu.VMEM((TB, B, H), f32),
            pltpu.VMEM((B, 128), f32),
        ],
        compiler_params=pltpu.CompilerParams(
            dimension_semantics=("arbitrary",),
            vmem_limit_bytes=56 * 1024 * 1024,
        ),
        name="no_execution_fused",
    )(spec, sc0, sc1, embedding.astype(bf16), wx, wo, whh, winit, wout, wd,
      we, b2(b_ih), b2(b_hh), b2(b_out), b2(b_init), b2(v_attn),
      ints(tokens_in), ints(tokens_out), ints(pointer_ids),
      ints(ptr_targets), ints(pointer_mask))
    return out[:, 0]
